# baseline (speedup 1.0000x reference)
import functools
import jax
import jax.numpy as jnp
from jax import lax
from jax.experimental import pallas as pl
from jax.experimental.pallas import tpu as pltpu
from jax.experimental.pallas import tpu_sc as plsc

_IM = 1024.0
_PAD = 4.0
_MIN_SIZE = 16.0
_PRE_N = 6000
_POST_N = 300
_NMS_T = 0.7
_N = 12288
_ROWS = 96
_COLS = 128

_NS = 16
_PER = _N // _NS
_CHUNKS = _PER // 16
_CPAD = _PER + 64


def _transform_body(c0_ref, c1_ref, d0_ref, d1_ref, d2_ref, d3_ref, d4_ref,
                    ax1_ref, ay1_ref, ax2_ref, ay2_ref, out_ref):
    shape = (_ROWS, _COLS)
    c0 = c0_ref[...]
    c1 = c1_ref[...]
    d0 = d0_ref[...]
    d1 = d1_ref[...]
    d2 = d2_ref[...]
    d3 = d3_ref[...]
    d4 = d4_ref[...]
    ax1 = ax1_ref[...]
    ay1 = ay1_ref[...]
    ax2 = ax2_ref[...]
    ay2 = ay2_ref[...]
    m = jnp.maximum(c0, c1)
    e0 = jnp.exp(c0 - m)
    e1 = jnp.exp(c1 - m)
    score = e1 / (e0 + e1)
    widths = ax2 - ax1 + 1.0
    heights = ay2 - ay1 + 1.0
    ctr_x = ax1 + 0.5 * widths
    ctr_y = ay1 + 0.5 * heights
    cx = d0 * widths + ctr_x
    cy = d1 * heights + ctr_y
    a = jnp.exp(d2) * widths * 0.5
    b = jnp.exp(d3) * heights * 0.5
    th = d4
    ct = jnp.cos(th)
    st = jnp.sin(th)
    hw = jnp.sqrt((a * ct) ** 2 + (b * st) ** 2) + _PAD
    hh = jnp.sqrt((a * st) ** 2 + (b * ct) ** 2) + _PAD
    x1 = jnp.clip(cx - hw, 0.0, _IM - 1.0)
    y1 = jnp.clip(cy - hh, 0.0, _IM - 1.0)
    x2 = jnp.clip(cx + hw, 0.0, _IM - 1.0)
    y2 = jnp.clip(cy + hh, 0.0, _IM - 1.0)
    ws = x2 - x1 + 1.0
    hs = y2 - y1 + 1.0
    valid = (ws >= _MIN_SIZE) & (hs >= _MIN_SIZE)
    score = jnp.where(valid, score, jnp.float32(-1e9))
    areas = ws * hs

    u = lax.bitcast_convert_type(score, jnp.int32)
    ordv = u ^ (lax.shift_right_arithmetic(u, 31) & jnp.int32(0x7FFFFFFF))
    lin = (lax.broadcasted_iota(jnp.int32, shape, 0) * _COLS
           + lax.broadcasted_iota(jnp.int32, shape, 1))

    def _bs1(_, lohi):
        lo, hi = lohi
        mid = (lo & hi) + ((lo ^ hi) >> 1)
        ge = jnp.sum((ordv >= mid).astype(jnp.int32)) >= _PRE_N
        return (jnp.where(ge, mid, lo), jnp.where(ge, hi, mid))

    tau, _ = lax.fori_loop(
        0, 32, _bs1, (jnp.int32(-2147483647 - 1), jnp.int32(2147483647)))

    n_greater = jnp.sum((ordv > tau).astype(jnp.int32))
    quota = _PRE_N - n_greater
    tie = ordv == tau

    def _bs2(_, lohi):
        lo, hi = lohi
        mid = (lo + hi) >> 1
        ge = jnp.sum((tie & (lin <= mid)).astype(jnp.int32)) >= quota
        return (jnp.where(ge, lo, mid), jnp.where(ge, mid, hi))

    _, idxcut = lax.fori_loop(0, 14, _bs2, (jnp.int32(-1), jnp.int32(_N - 1)))

    eligible = (ordv > tau) | (tie & (lin <= idxcut))
    live0 = jnp.where(eligible, score, jnp.float32(-jnp.inf))

    for k, v in enumerate((live0, x1, y1, x2, y2, cx, cy, a, b, th, score,
                           areas)):
        out_ref[k, :, :] = v


def _nms_sc(p_hbm, out_hbm, pl_v, rowbuf, postsl, frow, outbuf, compf,
            compi, posts_sh):
    ninf = jnp.float32(-jnp.inf)
    big_f = jnp.float32(1e30)
    big_i = jnp.int32(0x7FFFFFFF)
    w = lax.axis_index("s")
    cid = lax.axis_index("c")
    base = w * _PER
    ji = lax.iota(jnp.int32, 16)

    def splat_i(x):
        return jnp.zeros((16,), jnp.int32) + x

    def splat_f(x):
        return jnp.zeros((16,), jnp.float32) + x

    def splat_max(v):
        return plsc.cummax(jnp.flip(plsc.cummax(v), 0))

    def splat_min(v):
        return -splat_max(-v)

    pltpu.sync_copy(p_hbm.at[:, pl.ds(base, _PER)], pl_v)

    parlane = (ji >= 2) & (ji <= 12)
    rowsel = jnp.where(parlane, ji - 1, 0)

    def local_candidate(vmax, vidx):
        m_loc = splat_max(vmax)
        ili = splat_min(jnp.where(vmax == m_loc, vidx, big_i))
        i_loc = ili + base
        pv = plsc.load_gather(pl_v, [rowsel, ili])
        row = jnp.where(parlane, pv, jnp.zeros((16,), jnp.float32))
        row = jnp.where(ji == 0, m_loc, row)
        row = jnp.where(ji == 1, i_loc.astype(jnp.float32), row)
        rowbuf[...] = row

    for c in range(_CPAD // 16):
        compi[pl.ds(c * 16, 16)] = splat_i(jnp.int32(0))
    cnt = jnp.int32(0)
    for c in range(_CHUNKS):
        sl = pl.ds(c * 16, 16)
        msk = pl_v[0, sl] > ninf
        pos = plsc.cumsum(msk.astype(jnp.int32))
        dest = splat_i(cnt - 1) + pos
        plsc.store_scatter(compi, [dest], splat_i(c * 16) + ji, mask=msk)
        cnt = cnt + plsc.all_reduce_population_count(msk)[0]
    for c in range(_CPAD // 16):
        sl = pl.ds(c * 16, 16)
        idxv = compi[sl]
        lvc = plsc.load_gather(pl_v, [splat_i(jnp.int32(0)), idxv])
        pad = (splat_i(c * 16) + ji) >= cnt
        compf[0, sl] = jnp.where(pad, ninf, lvc)
        compf[1, sl] = plsc.load_gather(pl_v, [splat_i(jnp.int32(1)), idxv])
        compf[2, sl] = plsc.load_gather(pl_v, [splat_i(jnp.int32(2)), idxv])
        compf[3, sl] = plsc.load_gather(pl_v, [splat_i(jnp.int32(3)), idxv])
        compf[4, sl] = plsc.load_gather(pl_v, [splat_i(jnp.int32(4)), idxv])
        compf[5, sl] = plsc.load_gather(pl_v, [splat_i(jnp.int32(11)), idxv])
    ntrip = (cnt + 63) >> 6

    ymin1 = splat_f(jnp.float32(1e30))
    ymax2 = splat_f(jnp.float32(-1e30))
    for c in range(_CPAD // 16):
        sl = pl.ds(c * 16, 16)
        ymin1 = jnp.minimum(ymin1, compf[2, sl])
        ymax2 = jnp.maximum(ymax2, compf[4, sl])
    ymin1 = -splat_max(-ymin1)
    ymax2 = splat_max(ymax2)

    vmax = splat_f(ninf)
    vidx = splat_i(jnp.int32(0))
    for c in range(_CPAD // 16):
        sl = pl.ds(c * 16, 16)
        lv = compf[0, sl]
        gidx = compi[sl]
        upd = lv > vmax
        vmax = jnp.where(upd, lv, vmax)
        vidx = jnp.where(upd, gidx, vidx)
    local_candidate(vmax, vidx)

    permidx = jnp.where(ji < 10, ji + 2, 0)

    def body(i, carry):
        pltpu.sync_copy(rowbuf, posts_sh.at[i, w])
        plsc.subcore_barrier()
        pltpu.sync_copy(posts_sh.at[i], postsl)
        m_all = plsc.load_gather(postsl, [ji, splat_i(jnp.int32(0))])
        m_g = splat_max(m_all)
        wsel = splat_min(jnp.where(m_all == m_g, ji, jnp.int32(16)))

        outrow = plsc.load_gather(postsl, [wsel, permidx])
        outrow = jnp.where(ji < 10, outrow, jnp.float32(0.0))

        @pl.when(i == 0)
        def _():
            frow[...] = outrow

        rowf = jnp.where(m_g == ninf, frow[...], outrow)
        plsc.store_scatter(outbuf, [splat_i(i), ji], rowf)

        sy1 = plsc.load_gather(postsl, [wsel, splat_i(jnp.int32(3))])
        sy2 = plsc.load_gather(postsl, [wsel, splat_i(jnp.int32(5))])
        overlaps = jnp.logical_not((sy2[0] + 1.0 <= ymin1[0])
                                   | (sy1[0] >= ymax2[0] + 1.0))

        @pl.when(overlaps)
        def _suppress():
            sx1 = plsc.load_gather(postsl, [wsel, splat_i(jnp.int32(2))])
            sx2 = plsc.load_gather(postsl, [wsel, splat_i(jnp.int32(4))])
            sar = plsc.load_gather(postsl, [wsel, splat_i(jnp.int32(12))])

            def chunk4(t, vm_vi):
                vmax, vidx = vm_vi
                for u in range(4):
                    sl = pl.ds(t * 64 + u * 16, 16)
                    lv = compf[0, sl]
                    x1c = compf[1, sl]
                    y1c = compf[2, sl]
                    x2c = compf[3, sl]
                    y2c = compf[4, sl]
                    arc = compf[5, sl]
                    xx1 = jnp.maximum(x1c, sx1)
                    yy1 = jnp.maximum(y1c, sy1)
                    xx2 = jnp.minimum(x2c, sx2)
                    yy2 = jnp.minimum(y2c, sy2)
                    inter = (jnp.maximum(xx2 - xx1 + 1.0, 0.0)
                             * jnp.maximum(yy2 - yy1 + 1.0, 0.0))
                    iou = inter / (arc + sar - inter)
                    lv = jnp.where(iou > _NMS_T, ninf, lv)
                    compf[0, sl] = lv
                    gidx = compi[sl]
                    upd = lv > vmax
                    vmax = jnp.where(upd, lv, vmax)
                    vidx = jnp.where(upd, gidx, vidx)
                return vmax, vidx

            vmax, vidx = lax.fori_loop(
                0, ntrip, chunk4, (splat_f(ninf), splat_i(jnp.int32(0))))
            local_candidate(vmax, vidx)

        return carry

    lax.fori_loop(0, _POST_N, body, jnp.int32(0))

    @pl.when((w == 0) & (cid == 0))
    def _():
        pltpu.sync_copy(outbuf, out_hbm)


def kernel(out_cls, out_ellipse, anchors):
    c0 = out_cls[..., 0].reshape(_ROWS, _COLS)
    c1 = out_cls[..., 1].reshape(_ROWS, _COLS)
    ds = [out_ellipse[..., i].reshape(_ROWS, _COLS) for i in range(5)]
    axs = [anchors[:, i].reshape(_ROWS, _COLS) for i in range(4)]
    params = pl.pallas_call(
        _transform_body,
        out_shape=jax.ShapeDtypeStruct((12, _ROWS, _COLS), jnp.float32),
    )(c0, c1, *ds, *axs)
    params = params.reshape(12, _N)

    mesh = plsc.VectorSubcoreMesh(core_axis_name="c", subcore_axis_name="s",
                                  num_cores=1, num_subcores=_NS)
    nms = functools.partial(
        pl.kernel,
        out_type=jax.ShapeDtypeStruct((_POST_N, 16), jnp.float32),
        mesh=mesh,
        compiler_params=pltpu.CompilerParams(needs_layout_passes=False),
        scratch_types=[
            pltpu.VMEM((12, _PER), jnp.float32),
            pltpu.VMEM((16,), jnp.float32),
            pltpu.VMEM((16, 16), jnp.float32),
            pltpu.VMEM((16,), jnp.float32),
            pltpu.VMEM((_POST_N, 16), jnp.float32),
            pltpu.VMEM((6, _CPAD), jnp.float32),
            pltpu.VMEM((_CPAD,), jnp.int32),
            pltpu.VMEM_SHARED((_POST_N, 16, 16), jnp.float32),
        ],
    )(_nms_sc)
    out = nms(params)
    boxes = out[:, 0:4]
    ellipses = out[:, 4:9]
    scores = out[:, 9]
    return boxes, ellipses, scores


# speedup vs baseline: 1.0467x; 1.0467x over previous
import functools
import jax
import jax.numpy as jnp
from jax import lax
from jax.experimental import pallas as pl
from jax.experimental.pallas import tpu as pltpu
from jax.experimental.pallas import tpu_sc as plsc

_IM = 1024.0
_PAD = 4.0
_MIN_SIZE = 16.0
_PRE_N = 6000
_POST_N = 300
_NMS_T = 0.7
_N = 12288
_ROWS = 96
_COLS = 128

_NS = 16
_PER = _N // _NS
_CHUNKS = _PER // 16
_CPAD = _PER + 64


def _transform_body(c0_ref, c1_ref, d0_ref, d1_ref, d2_ref, d3_ref, d4_ref,
                    ax1_ref, ay1_ref, ax2_ref, ay2_ref, out_ref):
    shape = (_ROWS, _COLS)
    c0 = c0_ref[...]
    c1 = c1_ref[...]
    d0 = d0_ref[...]
    d1 = d1_ref[...]
    d2 = d2_ref[...]
    d3 = d3_ref[...]
    d4 = d4_ref[...]
    ax1 = ax1_ref[...]
    ay1 = ay1_ref[...]
    ax2 = ax2_ref[...]
    ay2 = ay2_ref[...]
    m = jnp.maximum(c0, c1)
    e0 = jnp.exp(c0 - m)
    e1 = jnp.exp(c1 - m)
    score = e1 / (e0 + e1)
    widths = ax2 - ax1 + 1.0
    heights = ay2 - ay1 + 1.0
    ctr_x = ax1 + 0.5 * widths
    ctr_y = ay1 + 0.5 * heights
    cx = d0 * widths + ctr_x
    cy = d1 * heights + ctr_y
    a = jnp.exp(d2) * widths * 0.5
    b = jnp.exp(d3) * heights * 0.5
    th = d4
    ct = jnp.cos(th)
    st = jnp.sin(th)
    hw = jnp.sqrt((a * ct) ** 2 + (b * st) ** 2) + _PAD
    hh = jnp.sqrt((a * st) ** 2 + (b * ct) ** 2) + _PAD
    x1 = jnp.clip(cx - hw, 0.0, _IM - 1.0)
    y1 = jnp.clip(cy - hh, 0.0, _IM - 1.0)
    x2 = jnp.clip(cx + hw, 0.0, _IM - 1.0)
    y2 = jnp.clip(cy + hh, 0.0, _IM - 1.0)
    ws = x2 - x1 + 1.0
    hs = y2 - y1 + 1.0
    valid = (ws >= _MIN_SIZE) & (hs >= _MIN_SIZE)
    score = jnp.where(valid, score, jnp.float32(-1e9))
    areas = ws * hs

    u = lax.bitcast_convert_type(score, jnp.int32)
    ordv = u ^ (lax.shift_right_arithmetic(u, 31) & jnp.int32(0x7FFFFFFF))
    lin = (lax.broadcasted_iota(jnp.int32, shape, 0) * _COLS
           + lax.broadcasted_iota(jnp.int32, shape, 1))

    def _bs1(_, lohi):
        lo, hi = lohi
        mid = (lo & hi) + ((lo ^ hi) >> 1)
        ge = jnp.sum((ordv >= mid).astype(jnp.int32)) >= _PRE_N
        return (jnp.where(ge, mid, lo), jnp.where(ge, hi, mid))

    tau, _ = lax.fori_loop(
        0, 32, _bs1, (jnp.int32(-2147483647 - 1), jnp.int32(2147483647)))

    n_greater = jnp.sum((ordv > tau).astype(jnp.int32))
    quota = _PRE_N - n_greater
    tie = ordv == tau

    def _bs2(_, lohi):
        lo, hi = lohi
        mid = (lo + hi) >> 1
        ge = jnp.sum((tie & (lin <= mid)).astype(jnp.int32)) >= quota
        return (jnp.where(ge, lo, mid), jnp.where(ge, mid, hi))

    _, idxcut = lax.fori_loop(0, 14, _bs2, (jnp.int32(-1), jnp.int32(_N - 1)))

    eligible = (ordv > tau) | (tie & (lin <= idxcut))
    live0 = jnp.where(eligible, score, jnp.float32(-jnp.inf))

    for k, v in enumerate((live0, x1, y1, x2, y2, cx, cy, a, b, th, score,
                           areas)):
        out_ref[k, :, :] = v


def _nms_sc(p_hbm, out_hbm, pl_v, rowbuf, postsl, frow, outbuf, compf,
            compi, posts_sh):
    ninf = jnp.float32(-jnp.inf)
    big_f = jnp.float32(1e30)
    big_i = jnp.int32(0x7FFFFFFF)
    w = lax.axis_index("s")
    cid = lax.axis_index("c")
    base = w * _PER
    ji = lax.iota(jnp.int32, 16)

    def splat_i(x):
        return jnp.zeros((16,), jnp.int32) + x

    def splat_f(x):
        return jnp.zeros((16,), jnp.float32) + x

    def splat_max(v):
        return plsc.cummax(jnp.flip(plsc.cummax(v), 0))

    def splat_min(v):
        return -splat_max(-v)

    pltpu.sync_copy(p_hbm.at[:, pl.ds(base, _PER)], pl_v)

    parlane = (ji >= 2) & (ji <= 12)
    rowsel = jnp.where(parlane, ji - 1, 0)

    def local_candidate(vmax, vidx):
        m_loc = splat_max(vmax)
        ili = splat_min(jnp.where(vmax == m_loc, vidx, big_i))
        i_loc = ili + base
        pv = plsc.load_gather(pl_v, [rowsel, ili])
        row = jnp.where(parlane, pv, jnp.zeros((16,), jnp.float32))
        row = jnp.where(ji == 0, m_loc, row)
        row = jnp.where(ji == 1, i_loc.astype(jnp.float32), row)
        rowbuf[...] = row

    for c in range(_CPAD // 16):
        compi[pl.ds(c * 16, 16)] = splat_i(jnp.int32(0))
    cnt = jnp.int32(0)
    for c in range(_CHUNKS):
        sl = pl.ds(c * 16, 16)
        msk = pl_v[0, sl] > ninf
        pos = plsc.cumsum(msk.astype(jnp.int32))
        dest = splat_i(cnt - 1) + pos
        plsc.store_scatter(compi, [dest], splat_i(c * 16) + ji, mask=msk)
        cnt = cnt + plsc.all_reduce_population_count(msk)[0]
    for c in range(_CPAD // 16):
        sl = pl.ds(c * 16, 16)
        idxv = compi[sl]
        lvc = plsc.load_gather(pl_v, [splat_i(jnp.int32(0)), idxv])
        pad = (splat_i(c * 16) + ji) >= cnt
        compf[0, sl] = jnp.where(pad, ninf, lvc)
        compf[1, sl] = plsc.load_gather(pl_v, [splat_i(jnp.int32(1)), idxv])
        compf[2, sl] = plsc.load_gather(pl_v, [splat_i(jnp.int32(2)), idxv])
        compf[3, sl] = plsc.load_gather(pl_v, [splat_i(jnp.int32(3)), idxv])
        compf[4, sl] = plsc.load_gather(pl_v, [splat_i(jnp.int32(4)), idxv])
        compf[5, sl] = plsc.load_gather(pl_v, [splat_i(jnp.int32(11)), idxv])
    ntrip = (cnt + 63) >> 6

    vmax = splat_f(ninf)
    vidx = splat_i(jnp.int32(0))
    for c in range(_CPAD // 16):
        sl = pl.ds(c * 16, 16)
        lv = compf[0, sl]
        gidx = compi[sl]
        upd = lv > vmax
        vmax = jnp.where(upd, lv, vmax)
        vidx = jnp.where(upd, gidx, vidx)
    local_candidate(vmax, vidx)

    permidx = jnp.where(ji < 10, ji + 2, 0)

    def body(i, carry):
        pltpu.sync_copy(rowbuf, posts_sh.at[i, w])
        plsc.subcore_barrier()
        pltpu.sync_copy(posts_sh.at[i], postsl)
        m_all = plsc.load_gather(postsl, [ji, splat_i(jnp.int32(0))])
        m_g = splat_max(m_all)
        wsel = splat_min(jnp.where(m_all == m_g, ji, jnp.int32(16)))

        outrow = plsc.load_gather(postsl, [wsel, permidx])
        outrow = jnp.where(ji < 10, outrow, jnp.float32(0.0))

        @pl.when(i == 0)
        def _():
            frow[...] = outrow

        rowf = jnp.where(m_g == ninf, frow[...], outrow)
        plsc.store_scatter(outbuf, [splat_i(i), ji], rowf)

        sx1 = plsc.load_gather(postsl, [wsel, splat_i(jnp.int32(2))])
        sy1 = plsc.load_gather(postsl, [wsel, splat_i(jnp.int32(3))])
        sx2 = plsc.load_gather(postsl, [wsel, splat_i(jnp.int32(4))])
        sy2 = plsc.load_gather(postsl, [wsel, splat_i(jnp.int32(5))])
        sar = plsc.load_gather(postsl, [wsel, splat_i(jnp.int32(12))])

        def chunk4(t, vm_vi):
            vmax, vidx = vm_vi
            for u in range(4):
                sl = pl.ds(t * 64 + u * 16, 16)
                lv = compf[0, sl]
                x1c = compf[1, sl]
                y1c = compf[2, sl]
                x2c = compf[3, sl]
                y2c = compf[4, sl]
                arc = compf[5, sl]
                xx1 = jnp.maximum(x1c, sx1)
                yy1 = jnp.maximum(y1c, sy1)
                xx2 = jnp.minimum(x2c, sx2)
                yy2 = jnp.minimum(y2c, sy2)
                inter = (jnp.maximum(xx2 - xx1 + 1.0, 0.0)
                         * jnp.maximum(yy2 - yy1 + 1.0, 0.0))
                iou = inter / (arc + sar - inter)
                lv = jnp.where(iou > _NMS_T, ninf, lv)
                compf[0, sl] = lv
                gidx = compi[sl]
                upd = lv > vmax
                vmax = jnp.where(upd, lv, vmax)
                vidx = jnp.where(upd, gidx, vidx)
            return vmax, vidx

        vmax, vidx = lax.fori_loop(
            0, ntrip, chunk4, (splat_f(ninf), splat_i(jnp.int32(0))))
        local_candidate(vmax, vidx)
        return carry

    lax.fori_loop(0, _POST_N, body, jnp.int32(0))

    @pl.when((w == 0) & (cid == 0))
    def _():
        pltpu.sync_copy(outbuf, out_hbm)


def kernel(out_cls, out_ellipse, anchors):
    c0 = out_cls[..., 0].reshape(_ROWS, _COLS)
    c1 = out_cls[..., 1].reshape(_ROWS, _COLS)
    ds = [out_ellipse[..., i].reshape(_ROWS, _COLS) for i in range(5)]
    axs = [anchors[:, i].reshape(_ROWS, _COLS) for i in range(4)]
    params = pl.pallas_call(
        _transform_body,
        out_shape=jax.ShapeDtypeStruct((12, _ROWS, _COLS), jnp.float32),
    )(c0, c1, *ds, *axs)
    params = params.reshape(12, _N)

    mesh = plsc.VectorSubcoreMesh(core_axis_name="c", subcore_axis_name="s",
                                  num_cores=1, num_subcores=_NS)
    nms = functools.partial(
        pl.kernel,
        out_type=jax.ShapeDtypeStruct((_POST_N, 16), jnp.float32),
        mesh=mesh,
        compiler_params=pltpu.CompilerParams(needs_layout_passes=False),
        scratch_types=[
            pltpu.VMEM((12, _PER), jnp.float32),
            pltpu.VMEM((16,), jnp.float32),
            pltpu.VMEM((16, 16), jnp.float32),
            pltpu.VMEM((16,), jnp.float32),
            pltpu.VMEM((_POST_N, 16), jnp.float32),
            pltpu.VMEM((6, _CPAD), jnp.float32),
            pltpu.VMEM((_CPAD,), jnp.int32),
            pltpu.VMEM_SHARED((_POST_N, 16, 16), jnp.float32),
        ],
    )(_nms_sc)
    out = nms(params)
    boxes = out[:, 0:4]
    ellipses = out[:, 4:9]
    scores = out[:, 9]
    return boxes, ellipses, scores


# speedup vs baseline: 1.0620x; 1.0146x over previous
import functools
import jax
import jax.numpy as jnp
from jax import lax
from jax.experimental import pallas as pl
from jax.experimental.pallas import tpu as pltpu
from jax.experimental.pallas import tpu_sc as plsc

_IM = 1024.0
_PAD = 4.0
_MIN_SIZE = 16.0
_PRE_N = 6000
_POST_N = 300
_NMS_T = 0.7
_N = 12288
_ROWS = 96
_COLS = 128

_NS = 16
_PER = _N // _NS
_CHUNKS = _PER // 16
_CPAD = _PER + 64


def _transform_body(c0_ref, c1_ref, d0_ref, d1_ref, d2_ref, d3_ref, d4_ref,
                    ax1_ref, ay1_ref, ax2_ref, ay2_ref, out_ref):
    shape = (_ROWS, _COLS)
    c0 = c0_ref[...]
    c1 = c1_ref[...]
    d0 = d0_ref[...]
    d1 = d1_ref[...]
    d2 = d2_ref[...]
    d3 = d3_ref[...]
    d4 = d4_ref[...]
    ax1 = ax1_ref[...]
    ay1 = ay1_ref[...]
    ax2 = ax2_ref[...]
    ay2 = ay2_ref[...]
    m = jnp.maximum(c0, c1)
    e0 = jnp.exp(c0 - m)
    e1 = jnp.exp(c1 - m)
    score = e1 / (e0 + e1)
    widths = ax2 - ax1 + 1.0
    heights = ay2 - ay1 + 1.0
    ctr_x = ax1 + 0.5 * widths
    ctr_y = ay1 + 0.5 * heights
    cx = d0 * widths + ctr_x
    cy = d1 * heights + ctr_y
    a = jnp.exp(d2) * widths * 0.5
    b = jnp.exp(d3) * heights * 0.5
    th = d4
    ct = jnp.cos(th)
    st = jnp.sin(th)
    hw = jnp.sqrt((a * ct) ** 2 + (b * st) ** 2) + _PAD
    hh = jnp.sqrt((a * st) ** 2 + (b * ct) ** 2) + _PAD
    x1 = jnp.clip(cx - hw, 0.0, _IM - 1.0)
    y1 = jnp.clip(cy - hh, 0.0, _IM - 1.0)
    x2 = jnp.clip(cx + hw, 0.0, _IM - 1.0)
    y2 = jnp.clip(cy + hh, 0.0, _IM - 1.0)
    ws = x2 - x1 + 1.0
    hs = y2 - y1 + 1.0
    valid = (ws >= _MIN_SIZE) & (hs >= _MIN_SIZE)
    score = jnp.where(valid, score, jnp.float32(-1e9))
    areas = ws * hs

    u = lax.bitcast_convert_type(score, jnp.int32)
    ordv = u ^ (lax.shift_right_arithmetic(u, 31) & jnp.int32(0x7FFFFFFF))
    lin = (lax.broadcasted_iota(jnp.int32, shape, 0) * _COLS
           + lax.broadcasted_iota(jnp.int32, shape, 1))

    def _bs1(_, lohi):
        lo, hi = lohi
        mid = (lo & hi) + ((lo ^ hi) >> 1)
        ge = jnp.sum((ordv >= mid).astype(jnp.int32)) >= _PRE_N
        return (jnp.where(ge, mid, lo), jnp.where(ge, hi, mid))

    tau, _ = lax.fori_loop(
        0, 32, _bs1, (jnp.int32(-2147483647 - 1), jnp.int32(2147483647)))

    n_greater = jnp.sum((ordv > tau).astype(jnp.int32))
    quota = _PRE_N - n_greater
    tie = ordv == tau

    def _bs2(_, lohi):
        lo, hi = lohi
        mid = (lo + hi) >> 1
        ge = jnp.sum((tie & (lin <= mid)).astype(jnp.int32)) >= quota
        return (jnp.where(ge, lo, mid), jnp.where(ge, mid, hi))

    _, idxcut = lax.fori_loop(0, 14, _bs2, (jnp.int32(-1), jnp.int32(_N - 1)))

    eligible = (ordv > tau) | (tie & (lin <= idxcut))
    live0 = jnp.where(eligible, score, jnp.float32(-jnp.inf))

    for k, v in enumerate((live0, x1, y1, x2, y2, cx, cy, a, b, th, score,
                           areas)):
        out_ref[k, :, :] = v


def _nms_sc(p_hbm, out_hbm, pl_v, rowbuf, postsl, frow, outbuf, compf,
            compi, posts_sh):
    ninf = jnp.float32(-jnp.inf)
    big_i = jnp.int32(0x7FFFFFFF)
    w = lax.axis_index("s")
    cid = lax.axis_index("c")
    base = w * _PER
    ji = lax.iota(jnp.int32, 16)

    def splat_i(x):
        return jnp.zeros((16,), jnp.int32) + x

    def splat_f(x):
        return jnp.zeros((16,), jnp.float32) + x

    def splat_max(v):
        return plsc.cummax(jnp.flip(plsc.cummax(v), 0))

    def splat_min(v):
        return -splat_max(-v)

    pltpu.sync_copy(p_hbm.at[:, pl.ds(base, _PER)], pl_v)

    parlane = (ji >= 2) & (ji <= 12)
    rowsel = jnp.where(parlane, ji - 1, 0)

    def local_candidate(vmax, vidx):
        m_loc = splat_max(vmax)
        ili = splat_min(jnp.where(vmax == m_loc, vidx, big_i))
        i_loc = ili + base
        pv = plsc.load_gather(pl_v, [rowsel, ili])
        row = jnp.where(parlane, pv, jnp.zeros((16,), jnp.float32))
        row = jnp.where(ji == 0, m_loc, row)
        row = jnp.where(ji == 1, i_loc.astype(jnp.float32), row)
        rowbuf[...] = row

    for c in range(_CPAD // 16):
        compi[pl.ds(c * 16, 16)] = splat_i(jnp.int32(0))
    cnt = jnp.int32(0)
    for c in range(_CHUNKS):
        sl = pl.ds(c * 16, 16)
        msk = pl_v[0, sl] > ninf
        pos = plsc.cumsum(msk.astype(jnp.int32))
        dest = splat_i(cnt - 1) + pos
        plsc.store_scatter(compi, [dest], splat_i(c * 16) + ji, mask=msk)
        cnt = cnt + plsc.all_reduce_population_count(msk)[0]
    for c in range(_CPAD // 16):
        sl = pl.ds(c * 16, 16)
        idxv = compi[sl]
        lvc = plsc.load_gather(pl_v, [splat_i(jnp.int32(0)), idxv])
        pad = (splat_i(c * 16) + ji) >= cnt
        compf[0, sl] = jnp.where(pad, ninf, lvc)
        compf[1, sl] = plsc.load_gather(pl_v, [splat_i(jnp.int32(1)), idxv])
        compf[2, sl] = plsc.load_gather(pl_v, [splat_i(jnp.int32(2)), idxv])
        compf[3, sl] = plsc.load_gather(pl_v, [splat_i(jnp.int32(3)), idxv])
        compf[4, sl] = plsc.load_gather(pl_v, [splat_i(jnp.int32(4)), idxv])
        compf[5, sl] = plsc.load_gather(pl_v, [splat_i(jnp.int32(11)), idxv])
    ntrip = (cnt + 63) >> 6

    vmax = splat_f(ninf)
    vidx = splat_i(jnp.int32(0))
    for c in range(_CPAD // 16):
        sl = pl.ds(c * 16, 16)
        lv = compf[0, sl]
        gidx = compi[sl]
        upd = lv > vmax
        vmax = jnp.where(upd, lv, vmax)
        vidx = jnp.where(upd, gidx, vidx)
    local_candidate(vmax, vidx)

    permidx = jnp.where(ji < 10, ji + 2, 0)

    def body(i, carry):
        pltpu.sync_copy(rowbuf, posts_sh.at[i, w])
        plsc.subcore_barrier()
        pltpu.sync_copy(posts_sh.at[i], postsl)
        m_all = plsc.load_gather(postsl, [ji, splat_i(jnp.int32(0))])
        m_g = splat_max(m_all)
        wsel = plsc.all_reduce_ffs(m_all == m_g)

        outrow = plsc.load_gather(postsl, [wsel, permidx])
        outrow = jnp.where(ji < 10, outrow, jnp.float32(0.0))

        @pl.when(i == 0)
        def _():
            frow[...] = outrow

        rowf = jnp.where(m_g == ninf, frow[...], outrow)
        plsc.store_scatter(outbuf, [splat_i(i), ji], rowf)

        sx1 = plsc.load_gather(postsl, [wsel, splat_i(jnp.int32(2))])
        sy1 = plsc.load_gather(postsl, [wsel, splat_i(jnp.int32(3))])
        sx2 = plsc.load_gather(postsl, [wsel, splat_i(jnp.int32(4))])
        sy2 = plsc.load_gather(postsl, [wsel, splat_i(jnp.int32(5))])
        sar = plsc.load_gather(postsl, [wsel, splat_i(jnp.int32(12))])

        def chunk4(t, vm_vi):
            vmax, vidx = vm_vi
            for u in range(4):
                sl = pl.ds(t * 64 + u * 16, 16)
                lv = compf[0, sl]
                x1c = compf[1, sl]
                y1c = compf[2, sl]
                x2c = compf[3, sl]
                y2c = compf[4, sl]
                arc = compf[5, sl]
                xx1 = jnp.maximum(x1c, sx1)
                yy1 = jnp.maximum(y1c, sy1)
                xx2 = jnp.minimum(x2c, sx2)
                yy2 = jnp.minimum(y2c, sy2)
                inter = (jnp.maximum(xx2 - xx1 + 1.0, 0.0)
                         * jnp.maximum(yy2 - yy1 + 1.0, 0.0))
                iou = inter / (arc + sar - inter)
                lv = jnp.where(iou > _NMS_T, ninf, lv)
                compf[0, sl] = lv
                gidx = compi[sl]
                upd = lv > vmax
                vmax = jnp.where(upd, lv, vmax)
                vidx = jnp.where(upd, gidx, vidx)
            return vmax, vidx

        vmax, vidx = lax.fori_loop(
            0, ntrip, chunk4, (splat_f(ninf), splat_i(jnp.int32(0))))
        local_candidate(vmax, vidx)
        return carry

    lax.fori_loop(0, _POST_N, body, jnp.int32(0))

    @pl.when((w == 0) & (cid == 0))
    def _():
        pltpu.sync_copy(outbuf, out_hbm)


def kernel(out_cls, out_ellipse, anchors):
    c0 = out_cls[..., 0].reshape(_ROWS, _COLS)
    c1 = out_cls[..., 1].reshape(_ROWS, _COLS)
    ds = [out_ellipse[..., i].reshape(_ROWS, _COLS) for i in range(5)]
    axs = [anchors[:, i].reshape(_ROWS, _COLS) for i in range(4)]
    params = pl.pallas_call(
        _transform_body,
        out_shape=jax.ShapeDtypeStruct((12, _ROWS, _COLS), jnp.float32),
    )(c0, c1, *ds, *axs)
    params = params.reshape(12, _N)

    mesh = plsc.VectorSubcoreMesh(core_axis_name="c", subcore_axis_name="s",
                                  num_cores=1, num_subcores=_NS)
    nms = functools.partial(
        pl.kernel,
        out_type=jax.ShapeDtypeStruct((_POST_N, 16), jnp.float32),
        mesh=mesh,
        compiler_params=pltpu.CompilerParams(needs_layout_passes=False),
        scratch_types=[
            pltpu.VMEM((12, _PER), jnp.float32),
            pltpu.VMEM((16,), jnp.float32),
            pltpu.VMEM((16, 16), jnp.float32),
            pltpu.VMEM((16,), jnp.float32),
            pltpu.VMEM((_POST_N, 16), jnp.float32),
            pltpu.VMEM((6, _CPAD), jnp.float32),
            pltpu.VMEM((_CPAD,), jnp.int32),
            pltpu.VMEM_SHARED((_POST_N, 16, 16), jnp.float32),
        ],
    )(_nms_sc)
    out = nms(params)
    boxes = out[:, 0:4]
    ellipses = out[:, 4:9]
    scores = out[:, 9]
    return boxes, ellipses, scores
